# baseline (speedup 1.0000x reference)
import jax
import jax.numpy as jnp
from jax import lax
from jax.experimental import pallas as pl
from jax.experimental.pallas import tpu as pltpu
from jax.experimental.pallas import tpu_sc as plsc

N = 10000
E = 320000
D = 128
S = 10
TAU = 0.5
LAMBDA = 1.0

NC = 2
NS = 16
NW = NC * NS
NP = 10240
EB = 80
EP = NW * EB * 128
GB = 25
GP = NW * GB * 128


def _sc_mesh():
    return plsc.VectorSubcoreMesh(core_axis_name="c", subcore_axis_name="s")


HD = D // 2
EB2 = 160
_EPH = 4
_EBP = EB2 // _EPH


def _segsum_body(x_hbm, src_hbm, dst_hbm, z_hbm, out_hbm,
                 xs, acc, srcv, dstv, rows0, rows1,
                 sem0, sem1, sct0, sct1, semz):
    c = lax.axis_index("c")
    s = lax.axis_index("s")
    rpt = NP // NS
    zcp = pltpu.async_copy(z_hbm.at[pl.ds(s * rpt, rpt)],
                           acc.at[pl.ds(s * rpt, rpt)], semz)
    pltpu.sync_copy(x_hbm.at[c, pl.ds(s * rpt, rpt)],
                    xs.at[pl.ds(s * rpt, rpt)])
    pltpu.sync_copy(src_hbm.at[s, pl.ds(0, _EBP)], srcv)
    pltpu.sync_copy(dst_hbm.at[s, pl.ds(0, _EBP)], dstv)
    zcp.wait()
    plsc.subcore_barrier()

    def phase(ph, carry):
        pltpu.async_copy(xs.at[srcv.at[0]], rows0, sem0)
        pltpu.async_copy(xs.at[srcv.at[1]], rows1, sem1)

        def body(jj, carry2):
            j0 = 2 * jj
            j1 = 2 * jj + 1
            j2 = jnp.minimum(2 * jj + 2, _EBP - 1)
            j3 = jnp.minimum(2 * jj + 3, _EBP - 1)
            pltpu.make_async_copy(xs.at[srcv.at[0]], rows0, sem0).wait()
            pltpu.async_copy(rows0, acc.at[dstv.at[j0]], sct0, add=True)
            pltpu.make_async_copy(xs.at[srcv.at[0]], rows1, sem1).wait()
            pltpu.async_copy(rows1, acc.at[dstv.at[j1]], sct1, add=True)
            pltpu.make_async_copy(rows0, acc.at[dstv.at[0]], sct0).wait()
            pltpu.async_copy(xs.at[srcv.at[j2]], rows0, sem0)
            pltpu.make_async_copy(rows1, acc.at[dstv.at[0]], sct1).wait()
            pltpu.async_copy(xs.at[srcv.at[j3]], rows1, sem1)
            return carry2

        lax.fori_loop(0, _EBP // 2, body, 0)
        pltpu.make_async_copy(xs.at[srcv.at[0]], rows0, sem0).wait()
        pltpu.make_async_copy(xs.at[srcv.at[0]], rows1, sem1).wait()
        nxt = jnp.minimum(ph + 1, _EPH - 1)
        pltpu.sync_copy(src_hbm.at[s, pl.ds(nxt * _EBP, _EBP)], srcv)
        pltpu.sync_copy(dst_hbm.at[s, pl.ds(nxt * _EBP, _EBP)], dstv)
        return carry

    lax.fori_loop(0, _EPH, phase, 0)
    plsc.subcore_barrier()
    pltpu.sync_copy(acc.at[pl.ds(s * rpt, rpt)],
                    out_hbm.at[c, pl.ds(s * rpt, rpt)])


def _segsum(x2, src3d, dst3d, zeros_half):
    kfn = pl.kernel(
        _segsum_body,
        out_type=jax.ShapeDtypeStruct((NC, NP, HD), jnp.float32),
        mesh=_sc_mesh(),
        scratch_types=[
            pltpu.VMEM_SHARED((NP, HD), jnp.float32),
            pltpu.VMEM_SHARED((NP, HD), jnp.float32),
            pltpu.VMEM((_EBP, 128), jnp.int32),
            pltpu.VMEM((_EBP, 128), jnp.int32),
            pltpu.VMEM((128, HD), jnp.float32),
            pltpu.VMEM((128, HD), jnp.float32),
            pltpu.SemaphoreType.DMA,
            pltpu.SemaphoreType.DMA,
            pltpu.SemaphoreType.DMA,
            pltpu.SemaphoreType.DMA,
            pltpu.SemaphoreType.DMA,
        ],
    )
    return kfn(x2, src3d, dst3d, zeros_half)


def _gather2_body(tab_hbm, ip_hbm, in_hbm, op_hbm, on_hbm,
                  ys, idxp, idxn, rows0, rows1, sem0, sem1, swr0, swr1):
    c = lax.axis_index("c")
    s = lax.axis_index("s")
    wid = s * NC + c
    @pl.when(s < NS - 1)
    def _():
        pltpu.sync_copy(tab_hbm.at[pl.ds(s * 624, 624)],
                        ys.at[pl.ds(s * 624, 624)])

    @pl.when(s == NS - 1)
    def _():
        pltpu.sync_copy(tab_hbm.at[pl.ds(624 * (NS - 1), N - 624 * (NS - 1))],
                        ys.at[pl.ds(624 * (NS - 1), N - 624 * (NS - 1))])

    pltpu.sync_copy(ip_hbm.at[wid], idxp)
    pltpu.sync_copy(in_hbm.at[wid], idxn)
    plsc.subcore_barrier()
    pltpu.async_copy(ys.at[idxp.at[0]], rows0, sem0)
    pltpu.async_copy(ys.at[idxn.at[0]], rows1, sem1)

    def body(j, carry):
        g = wid * GB + j
        jn = jnp.minimum(j + 1, GB - 1)
        pltpu.make_async_copy(ys.at[idxp.at[0]], rows0, sem0).wait()
        pltpu.async_copy(rows0, op_hbm.at[pl.ds(g * 128, 128)], swr0)
        pltpu.make_async_copy(ys.at[idxn.at[0]], rows1, sem1).wait()
        pltpu.async_copy(rows1, on_hbm.at[pl.ds(g * 128, 128)], swr1)
        pltpu.make_async_copy(rows0, op_hbm.at[pl.ds(0, 128)], swr0).wait()
        pltpu.async_copy(ys.at[idxp.at[jn]], rows0, sem0)
        pltpu.make_async_copy(rows1, on_hbm.at[pl.ds(0, 128)], swr1).wait()
        pltpu.async_copy(ys.at[idxn.at[jn]], rows1, sem1)
        return carry

    lax.fori_loop(0, GB, body, 0)
    pltpu.make_async_copy(ys.at[idxp.at[0]], rows0, sem0).wait()
    pltpu.make_async_copy(ys.at[idxn.at[0]], rows1, sem1).wait()


def _gather2(tab, idxp3d, idxn3d):
    kfn = pl.kernel(
        _gather2_body,
        out_type=(jax.ShapeDtypeStruct((GP, D), jnp.float32),
                  jax.ShapeDtypeStruct((GP, D), jnp.float32)),
        mesh=_sc_mesh(),
        scratch_types=[
            pltpu.VMEM_SHARED((NP, D), jnp.float32),
            pltpu.VMEM((GB, 128), jnp.int32),
            pltpu.VMEM((GB, 128), jnp.int32),
            pltpu.VMEM((128, D), jnp.float32),
            pltpu.VMEM((128, D), jnp.float32),
            pltpu.SemaphoreType.DMA,
            pltpu.SemaphoreType.DMA,
            pltpu.SemaphoreType.DMA,
            pltpu.SemaphoreType.DMA,
        ],
    )
    return kfn(tab, idxp3d, idxn3d)


_BN = 1024


def _gin_hidden(x_ref, p_ref, e_ref, wat_ref, wab_ref, ba_ref):
    ul = x_ref[0] * (1.0 + e_ref[...]) + p_ref[0]
    ur = x_ref[1] * (1.0 + e_ref[...]) + p_ref[1]
    return jnp.maximum(
        jnp.dot(ul, wat_ref[...], preferred_element_type=jnp.float32)
        + jnp.dot(ur, wab_ref[...], preferred_element_type=jnp.float32)
        + ba_ref[...], 0.0)


def _gin_mlp_kernel(x_ref, p_ref, e_ref, wat_ref, wab_ref, ba_ref,
                    wb_ref, bb_ref, o_ref):
    t = _gin_hidden(x_ref, p_ref, e_ref, wat_ref, wab_ref, ba_ref)
    y = (jnp.dot(t, wb_ref[...], preferred_element_type=jnp.float32)
         + bb_ref[...])
    o_ref[0] = y[:, :HD]
    o_ref[1] = y[:, HD:]


def _gin_mlp(x2, parts, eps_row, WaT, WaB, ba, Wb, bb):
    wspec = pl.BlockSpec((D, D), lambda b: (0, 0))
    hspec = pl.BlockSpec((HD, D), lambda b: (0, 0))
    bspec = pl.BlockSpec((1, D), lambda b: (0, 0))
    espec = pl.BlockSpec((1, HD), lambda b: (0, 0))
    h2spec = pl.BlockSpec((2, _BN, HD), lambda b: (0, b, 0))
    return pl.pallas_call(
        _gin_mlp_kernel,
        grid=(NP // _BN,),
        in_specs=[h2spec, h2spec, espec, hspec, hspec, bspec, wspec, bspec],
        out_specs=h2spec,
        out_shape=jax.ShapeDtypeStruct((2, NP, HD), jnp.float32),
    )(x2, parts, eps_row, WaT, WaB, ba, Wb, bb)


def _l2n(y):
    n = jnp.sqrt(jnp.sum(y * y, axis=1, keepdims=True))
    return y / jnp.maximum(n, 1e-12)


_BM = 400


def _gin_mlp2_kernel(x_ref, p_ref, e_ref, wat_ref, wab_ref, ba_ref, wb_ref,
                     bb_ref, wp1_ref, bp1_ref, wp2_ref, bp2_ref, wp3_ref,
                     bp3_ref, y_ref, yn_ref, q_ref):
    t = _gin_hidden(x_ref, p_ref, e_ref, wat_ref, wab_ref, ba_ref)
    y = (jnp.dot(t, wb_ref[...], preferred_element_type=jnp.float32)
         + bb_ref[...])
    y_ref[...] = y
    yn_ref[...] = _l2n(y)
    q = jnp.maximum(
        jnp.dot(y, wp1_ref[...], preferred_element_type=jnp.float32)
        + bp1_ref[...], 0.0)
    q = jnp.maximum(
        jnp.dot(q, wp2_ref[...], preferred_element_type=jnp.float32)
        + bp2_ref[...], 0.0)
    q = (jnp.dot(q, wp3_ref[...], preferred_element_type=jnp.float32)
         + bp3_ref[...])
    q_ref[...] = _l2n(q)


def _gin_mlp2(x2, parts, eps_row, WaT, WaB, ba, Wb, bb,
              Wp1, bp1, Wp2, bp2, Wp3, bp3):
    wspec = pl.BlockSpec((D, D), lambda b: (0, 0))
    hspec = pl.BlockSpec((HD, D), lambda b: (0, 0))
    bspec = pl.BlockSpec((1, D), lambda b: (0, 0))
    espec = pl.BlockSpec((1, HD), lambda b: (0, 0))
    h2spec = pl.BlockSpec((2, _BM, HD), lambda b: (0, b, 0))
    rspec = pl.BlockSpec((_BM, D), lambda b: (b, 0))
    sds = jax.ShapeDtypeStruct((N, D), jnp.float32)
    return pl.pallas_call(
        _gin_mlp2_kernel,
        grid=(N // _BM,),
        in_specs=[
            h2spec, h2spec,
            espec, hspec, hspec, bspec, wspec, bspec,
            wspec, bspec, wspec, bspec, wspec, bspec,
        ],
        out_specs=(rspec, rspec, rspec),
        out_shape=(sds, sds, sds),
    )(x2, parts, eps_row, WaT, WaB, ba, Wb, bb,
      Wp1, bp1, Wp2, bp2, Wp3, bp3)


_BL = 400


def _loss_kernel(*refs):
    pr_ref, on_ref = refs[0], refs[1]
    pos_refs = refs[2:2 + S]
    neg_refs = refs[2 + S:2 + 2 * S]
    loss_ref = refs[-1]
    pr = pr_ref[...]
    on = on_ref[...]
    ps = jnp.zeros((_BL, 1), jnp.float32)
    ns = jnp.zeros((_BL, 1), jnp.float32)
    for s in range(S):
        pd = jnp.sum(pr * pos_refs[s][0], axis=1, keepdims=True)
        nd = jnp.sum(on * neg_refs[s][0], axis=1, keepdims=True)
        ps = ps + jnp.exp(pd / TAU)
        ns = ns + jnp.exp(nd / TAU)
    part = -jnp.sum(jnp.log(ps / (ps + LAMBDA * ns))) / N

    @pl.when(pl.program_id(0) == 0)
    def _():
        loss_ref[0, 0] = 0.0

    loss_ref[0, 0] += part


def _loss(proj, out_norm, pos3, neg3):
    rspec = pl.BlockSpec((_BL, D), lambda b: (b, 0))
    in_specs = [rspec, rspec]
    for s in range(S):
        in_specs.append(pl.BlockSpec((1, _BL, D), lambda b, s=s: (s, b, 0)))
    for s in range(S):
        in_specs.append(pl.BlockSpec((1, _BL, D), lambda b, s=s: (s, b, 0)))
    return pl.pallas_call(
        _loss_kernel,
        grid=(N // _BL,),
        in_specs=in_specs,
        out_specs=pl.BlockSpec((1, 1), lambda b: (0, 0),
                               memory_space=pltpu.SMEM),
        out_shape=jax.ShapeDtypeStruct((1, 1), jnp.float32),
    )(proj, out_norm, *([pos3] * S + [neg3] * S))


def _prep_gather_idx(idx):
    t = jnp.transpose(idx.astype(jnp.int32))
    t = jnp.concatenate(
        [t, jnp.zeros((S, NP - N), jnp.int32)], axis=1)
    return t.reshape(NW, GB, 128)


def kernel(h, edge_index, nbr_idx, neg_idx,
           eps1, W1a, b1a, W1b, b1b,
           eps2, W2a, b2a, W2b, b2b,
           Wp1, bp1, Wp2, bp2, Wp3, bp3):
    src = edge_index[0].astype(jnp.int32)
    dst = edge_index[1].astype(jnp.int32)
    pad_s = jnp.zeros((EP - E,), jnp.int32)
    pad_d = jnp.full((EP - E,), NP - 1, jnp.int32)
    src3d = jnp.concatenate([src, pad_s]).reshape(NS, EB2, 128)
    dst3d = jnp.concatenate([dst, pad_d]).reshape(NS, EB2, 128)
    zeros_half = jnp.zeros((NP, HD), jnp.float32)
    rpad = jnp.zeros((NP - N, HD), jnp.float32)
    h2 = jnp.stack([jnp.concatenate([h[:, :HD], rpad]),
                    jnp.concatenate([h[:, HD:], rpad])])
    W1aT, W1aB = W1a[:HD], W1a[HD:]
    W2aT, W2aB = W2a[:HD], W2a[HD:]
    e1 = jnp.broadcast_to(jnp.reshape(eps1, (1, 1)), (1, HD))
    e2 = jnp.broadcast_to(jnp.reshape(eps2, (1, 1)), (1, HD))
    b1a2, b1b2 = b1a.reshape(1, D), b1b.reshape(1, D)
    b2a2, b2b2 = b2a.reshape(1, D), b2b.reshape(1, D)
    bp12, bp22, bp32 = bp1.reshape(1, D), bp2.reshape(1, D), bp3.reshape(1, D)
    idxp3d = _prep_gather_idx(nbr_idx)
    idxn3d = _prep_gather_idx(neg_idx)

    agg1 = _segsum(h2, src3d, dst3d, zeros_half)
    l12 = _gin_mlp(h2, agg1, e1, W1aT, W1aB, b1a2, W1b, b1b2)

    agg2 = _segsum(l12, src3d, dst3d, zeros_half)
    y, yn, q = _gin_mlp2(l12, agg2, e2, W2aT, W2aB, b2a2, W2b, b2b2,
                         Wp1, bp12, Wp2, bp22, Wp3, bp32)

    pos_rows, neg_rows = _gather2(yn, idxp3d, idxn3d)
    pos3 = pos_rows.reshape(S, NP, D)
    neg3 = neg_rows.reshape(S, NP, D)
    loss2d = _loss(q, yn, pos3, neg3)

    loss = loss2d.reshape(())
    return loss, y


# speedup vs baseline: 1.0158x; 1.0158x over previous
import jax
import jax.numpy as jnp
from jax import lax
from jax.experimental import pallas as pl
from jax.experimental.pallas import tpu as pltpu
from jax.experimental.pallas import tpu_sc as plsc

N = 10000
E = 320000
D = 128
S = 10
TAU = 0.5
LAMBDA = 1.0

NC = 2
NS = 16
NW = NC * NS
NP = 10240
EB = 80
EP = NW * EB * 128
GB = 25
GP = NW * GB * 128


def _sc_mesh():
    return plsc.VectorSubcoreMesh(core_axis_name="c", subcore_axis_name="s")


HD = D // 2
EB2 = 160
_EPH = 4
_EBP = EB2 // _EPH


def _segsum_body(x_hbm, src_hbm, dst_hbm, z_hbm, out_hbm,
                 xs, acc, srcv, dstv, rows0, rows1,
                 sem0, sem1, sct0, sct1, semz):
    c = lax.axis_index("c")
    s = lax.axis_index("s")
    rpt = NP // NS
    zcp = pltpu.async_copy(z_hbm.at[pl.ds(s * rpt, rpt)],
                           acc.at[pl.ds(s * rpt, rpt)], semz)
    pltpu.sync_copy(x_hbm.at[c, pl.ds(s * rpt, rpt)],
                    xs.at[pl.ds(s * rpt, rpt)])
    pltpu.sync_copy(src_hbm.at[s, pl.ds(0, _EBP)], srcv)
    pltpu.sync_copy(dst_hbm.at[s, pl.ds(0, _EBP)], dstv)
    zcp.wait()
    plsc.subcore_barrier()

    def phase(ph, carry):
        pltpu.async_copy(xs.at[srcv.at[0]], rows0, sem0)
        pltpu.async_copy(xs.at[srcv.at[1]], rows1, sem1)

        def body(jj, carry2):
            j0 = 2 * jj
            j1 = 2 * jj + 1
            j2 = jnp.minimum(2 * jj + 2, _EBP - 1)
            j3 = jnp.minimum(2 * jj + 3, _EBP - 1)
            pltpu.make_async_copy(xs.at[srcv.at[0]], rows0, sem0).wait()
            pltpu.async_copy(rows0, acc.at[dstv.at[j0]], sct0, add=True)
            pltpu.make_async_copy(xs.at[srcv.at[0]], rows1, sem1).wait()
            pltpu.async_copy(rows1, acc.at[dstv.at[j1]], sct1, add=True)
            pltpu.make_async_copy(rows0, acc.at[dstv.at[0]], sct0).wait()
            pltpu.async_copy(xs.at[srcv.at[j2]], rows0, sem0)
            pltpu.make_async_copy(rows1, acc.at[dstv.at[0]], sct1).wait()
            pltpu.async_copy(xs.at[srcv.at[j3]], rows1, sem1)
            return carry2

        lax.fori_loop(0, _EBP // 2, body, 0)
        pltpu.make_async_copy(xs.at[srcv.at[0]], rows0, sem0).wait()
        pltpu.make_async_copy(xs.at[srcv.at[0]], rows1, sem1).wait()
        nxt = jnp.minimum(ph + 1, _EPH - 1)
        pltpu.sync_copy(src_hbm.at[s, pl.ds(nxt * _EBP, _EBP)], srcv)
        pltpu.sync_copy(dst_hbm.at[s, pl.ds(nxt * _EBP, _EBP)], dstv)
        return carry

    lax.fori_loop(0, _EPH, phase, 0)
    plsc.subcore_barrier()
    pltpu.sync_copy(acc.at[pl.ds(s * rpt, rpt)],
                    out_hbm.at[c, pl.ds(s * rpt, rpt)])


def _segsum(x2, src3d, dst3d, zeros_half):
    kfn = pl.kernel(
        _segsum_body,
        out_type=jax.ShapeDtypeStruct((NC, NP, HD), jnp.float32),
        mesh=_sc_mesh(),
        scratch_types=[
            pltpu.VMEM_SHARED((NP, HD), jnp.float32),
            pltpu.VMEM_SHARED((NP, HD), jnp.float32),
            pltpu.VMEM((_EBP, 128), jnp.int32),
            pltpu.VMEM((_EBP, 128), jnp.int32),
            pltpu.VMEM((128, HD), jnp.float32),
            pltpu.VMEM((128, HD), jnp.float32),
            pltpu.SemaphoreType.DMA,
            pltpu.SemaphoreType.DMA,
            pltpu.SemaphoreType.DMA,
            pltpu.SemaphoreType.DMA,
            pltpu.SemaphoreType.DMA,
        ],
    )
    return kfn(x2, src3d, dst3d, zeros_half)


def _gather2_body(tab_hbm, ip_hbm, in_hbm, op_hbm, on_hbm,
                  ys, idxp, idxn, rows0, rows1, sem0, sem1, swr0, swr1):
    c = lax.axis_index("c")
    s = lax.axis_index("s")
    wid = s * NC + c
    rpt = NP // NS
    pltpu.sync_copy(tab_hbm.at[pl.ds(s * rpt, rpt)],
                    ys.at[pl.ds(s * rpt, rpt)])
    pltpu.sync_copy(ip_hbm.at[wid], idxp)
    pltpu.sync_copy(in_hbm.at[wid], idxn)
    plsc.subcore_barrier()
    pltpu.async_copy(ys.at[idxp.at[0]], rows0, sem0)
    pltpu.async_copy(ys.at[idxn.at[0]], rows1, sem1)

    def body(j, carry):
        g = wid * GB + j
        jn = jnp.minimum(j + 1, GB - 1)
        pltpu.make_async_copy(ys.at[idxp.at[0]], rows0, sem0).wait()
        pltpu.async_copy(rows0, op_hbm.at[pl.ds(g * 128, 128)], swr0)
        pltpu.make_async_copy(ys.at[idxn.at[0]], rows1, sem1).wait()
        pltpu.async_copy(rows1, on_hbm.at[pl.ds(g * 128, 128)], swr1)
        pltpu.make_async_copy(rows0, op_hbm.at[pl.ds(0, 128)], swr0).wait()
        pltpu.async_copy(ys.at[idxp.at[jn]], rows0, sem0)
        pltpu.make_async_copy(rows1, on_hbm.at[pl.ds(0, 128)], swr1).wait()
        pltpu.async_copy(ys.at[idxn.at[jn]], rows1, sem1)
        return carry

    lax.fori_loop(0, GB, body, 0)
    pltpu.make_async_copy(ys.at[idxp.at[0]], rows0, sem0).wait()
    pltpu.make_async_copy(ys.at[idxn.at[0]], rows1, sem1).wait()


def _gather2(tab, idxp3d, idxn3d):
    kfn = pl.kernel(
        _gather2_body,
        out_type=(jax.ShapeDtypeStruct((GP, D), jnp.float32),
                  jax.ShapeDtypeStruct((GP, D), jnp.float32)),
        mesh=_sc_mesh(),
        scratch_types=[
            pltpu.VMEM_SHARED((NP, D), jnp.float32),
            pltpu.VMEM((GB, 128), jnp.int32),
            pltpu.VMEM((GB, 128), jnp.int32),
            pltpu.VMEM((128, D), jnp.float32),
            pltpu.VMEM((128, D), jnp.float32),
            pltpu.SemaphoreType.DMA,
            pltpu.SemaphoreType.DMA,
            pltpu.SemaphoreType.DMA,
            pltpu.SemaphoreType.DMA,
        ],
    )
    return kfn(tab, idxp3d, idxn3d)


_BN = 1024


def _gin_hidden(x_ref, p_ref, e_ref, wat_ref, wab_ref, ba_ref):
    ul = x_ref[0] * (1.0 + e_ref[...]) + p_ref[0]
    ur = x_ref[1] * (1.0 + e_ref[...]) + p_ref[1]
    return jnp.maximum(
        jnp.dot(ul, wat_ref[...], preferred_element_type=jnp.float32)
        + jnp.dot(ur, wab_ref[...], preferred_element_type=jnp.float32)
        + ba_ref[...], 0.0)


def _gin_mlp_kernel(x_ref, p_ref, e_ref, wat_ref, wab_ref, ba_ref,
                    wb_ref, bb_ref, o_ref):
    t = _gin_hidden(x_ref, p_ref, e_ref, wat_ref, wab_ref, ba_ref)
    y = (jnp.dot(t, wb_ref[...], preferred_element_type=jnp.float32)
         + bb_ref[...])
    o_ref[0] = y[:, :HD]
    o_ref[1] = y[:, HD:]


def _gin_mlp(x2, parts, eps_row, WaT, WaB, ba, Wb, bb):
    wspec = pl.BlockSpec((D, D), lambda b: (0, 0))
    hspec = pl.BlockSpec((HD, D), lambda b: (0, 0))
    bspec = pl.BlockSpec((1, D), lambda b: (0, 0))
    espec = pl.BlockSpec((1, HD), lambda b: (0, 0))
    h2spec = pl.BlockSpec((2, _BN, HD), lambda b: (0, b, 0))
    return pl.pallas_call(
        _gin_mlp_kernel,
        grid=(NP // _BN,),
        in_specs=[h2spec, h2spec, espec, hspec, hspec, bspec, wspec, bspec],
        out_specs=h2spec,
        out_shape=jax.ShapeDtypeStruct((2, NP, HD), jnp.float32),
    )(x2, parts, eps_row, WaT, WaB, ba, Wb, bb)


def _l2n(y):
    n = jnp.sqrt(jnp.sum(y * y, axis=1, keepdims=True))
    return y / jnp.maximum(n, 1e-12)


def _gin_mlp2_kernel(x_ref, p_ref, e_ref, wat_ref, wab_ref, ba_ref, wb_ref,
                     bb_ref, wp1_ref, bp1_ref, wp2_ref, bp2_ref, wp3_ref,
                     bp3_ref, y_ref, yn_ref, q_ref):
    t = _gin_hidden(x_ref, p_ref, e_ref, wat_ref, wab_ref, ba_ref)
    y = (jnp.dot(t, wb_ref[...], preferred_element_type=jnp.float32)
         + bb_ref[...])
    y_ref[...] = y
    yn_ref[...] = _l2n(y)
    q = jnp.maximum(
        jnp.dot(y, wp1_ref[...], preferred_element_type=jnp.float32)
        + bp1_ref[...], 0.0)
    q = jnp.maximum(
        jnp.dot(q, wp2_ref[...], preferred_element_type=jnp.float32)
        + bp2_ref[...], 0.0)
    q = (jnp.dot(q, wp3_ref[...], preferred_element_type=jnp.float32)
         + bp3_ref[...])
    q_ref[...] = _l2n(q)


def _gin_mlp2(x2, parts, eps_row, WaT, WaB, ba, Wb, bb,
              Wp1, bp1, Wp2, bp2, Wp3, bp3):
    wspec = pl.BlockSpec((D, D), lambda b: (0, 0))
    hspec = pl.BlockSpec((HD, D), lambda b: (0, 0))
    bspec = pl.BlockSpec((1, D), lambda b: (0, 0))
    espec = pl.BlockSpec((1, HD), lambda b: (0, 0))
    h2spec = pl.BlockSpec((2, _BN, HD), lambda b: (0, b, 0))
    rspec = pl.BlockSpec((_BN, D), lambda b: (b, 0))
    sds = jax.ShapeDtypeStruct((NP, D), jnp.float32)
    return pl.pallas_call(
        _gin_mlp2_kernel,
        grid=(NP // _BN,),
        in_specs=[
            h2spec, h2spec,
            espec, hspec, hspec, bspec, wspec, bspec,
            wspec, bspec, wspec, bspec, wspec, bspec,
        ],
        out_specs=(rspec, rspec, rspec),
        out_shape=(sds, sds, sds),
    )(x2, parts, eps_row, WaT, WaB, ba, Wb, bb,
      Wp1, bp1, Wp2, bp2, Wp3, bp3)


_BL = 400


def _loss_kernel(*refs):
    pr_ref, on_ref = refs[0], refs[1]
    pos_refs = refs[2:2 + S]
    neg_refs = refs[2 + S:2 + 2 * S]
    loss_ref = refs[-1]
    pr = pr_ref[...]
    on = on_ref[...]
    ps = jnp.zeros((_BL, 1), jnp.float32)
    ns = jnp.zeros((_BL, 1), jnp.float32)
    for s in range(S):
        pd = jnp.sum(pr * pos_refs[s][0], axis=1, keepdims=True)
        nd = jnp.sum(on * neg_refs[s][0], axis=1, keepdims=True)
        ps = ps + jnp.exp(pd / TAU)
        ns = ns + jnp.exp(nd / TAU)
    part = -jnp.sum(jnp.log(ps / (ps + LAMBDA * ns))) / N

    @pl.when(pl.program_id(0) == 0)
    def _():
        loss_ref[0, 0] = 0.0

    loss_ref[0, 0] += part


def _loss(proj, out_norm, pos3, neg3):
    rspec = pl.BlockSpec((_BL, D), lambda b: (b, 0))
    in_specs = [rspec, rspec]
    for s in range(S):
        in_specs.append(pl.BlockSpec((1, _BL, D), lambda b, s=s: (s, b, 0)))
    for s in range(S):
        in_specs.append(pl.BlockSpec((1, _BL, D), lambda b, s=s: (s, b, 0)))
    return pl.pallas_call(
        _loss_kernel,
        grid=(N // _BL,),
        in_specs=in_specs,
        out_specs=pl.BlockSpec((1, 1), lambda b: (0, 0),
                               memory_space=pltpu.SMEM),
        out_shape=jax.ShapeDtypeStruct((1, 1), jnp.float32),
    )(proj, out_norm, *([pos3] * S + [neg3] * S))


def _prep_gather_idx(idx):
    t = jnp.transpose(idx.astype(jnp.int32))
    t = jnp.concatenate(
        [t, jnp.zeros((S, NP - N), jnp.int32)], axis=1)
    return t.reshape(NW, GB, 128)


def kernel(h, edge_index, nbr_idx, neg_idx,
           eps1, W1a, b1a, W1b, b1b,
           eps2, W2a, b2a, W2b, b2b,
           Wp1, bp1, Wp2, bp2, Wp3, bp3):
    src = edge_index[0].astype(jnp.int32)
    dst = edge_index[1].astype(jnp.int32)
    pad_s = jnp.zeros((EP - E,), jnp.int32)
    pad_d = jnp.full((EP - E,), NP - 1, jnp.int32)
    src3d = jnp.concatenate([src, pad_s]).reshape(NS, EB2, 128)
    dst3d = jnp.concatenate([dst, pad_d]).reshape(NS, EB2, 128)
    zeros_half = jnp.zeros((NP, HD), jnp.float32)
    rpad = jnp.zeros((NP - N, HD), jnp.float32)
    h2 = jnp.stack([jnp.concatenate([h[:, :HD], rpad]),
                    jnp.concatenate([h[:, HD:], rpad])])
    W1aT, W1aB = W1a[:HD], W1a[HD:]
    W2aT, W2aB = W2a[:HD], W2a[HD:]
    e1 = jnp.broadcast_to(jnp.reshape(eps1, (1, 1)), (1, HD))
    e2 = jnp.broadcast_to(jnp.reshape(eps2, (1, 1)), (1, HD))
    b1a2, b1b2 = b1a.reshape(1, D), b1b.reshape(1, D)
    b2a2, b2b2 = b2a.reshape(1, D), b2b.reshape(1, D)
    bp12, bp22, bp32 = bp1.reshape(1, D), bp2.reshape(1, D), bp3.reshape(1, D)
    idxp3d = _prep_gather_idx(nbr_idx)
    idxn3d = _prep_gather_idx(neg_idx)

    agg1 = _segsum(h2, src3d, dst3d, zeros_half)
    l12 = _gin_mlp(h2, agg1, e1, W1aT, W1aB, b1a2, W1b, b1b2)

    agg2 = _segsum(l12, src3d, dst3d, zeros_half)
    y, yn, q = _gin_mlp2(l12, agg2, e2, W2aT, W2aB, b2a2, W2b, b2b2,
                         Wp1, bp12, Wp2, bp22, Wp3, bp32)

    pos_rows, neg_rows = _gather2(yn, idxp3d, idxn3d)
    pos3 = pos_rows.reshape(S, NP, D)
    neg3 = neg_rows.reshape(S, NP, D)
    loss2d = _loss(q, yn, pos3, neg3)

    loss = loss2d.reshape(())
    hij = y[:N]
    return loss, hij
